# Initial kernel scaffold; baseline (speedup 1.0000x reference)
import jax
import jax.numpy as jnp
from jax.experimental import pallas as pl


def kernel(train, val, test, info_cat, idxs, init_lev_sms, init_seas_sms, init_seasonalities):
    raise NotImplementedError("write your pallas kernel here")


# speedup vs baseline: 35.9974x; 35.9974x over previous
import functools

import jax
import jax.numpy as jnp
from jax import lax
from jax.experimental import pallas as pl
from jax.experimental.pallas import tpu as pltpu
from jax.experimental.pallas import tpu_sc as plsc


def _sc_gather(idxs, lev_tab, seas_tab, rows_flat):
    info = plsc.get_sparse_core_info()
    nc = info.num_cores
    nw = nc * info.num_subcores
    b = idxs.shape[0]
    bpw = b // nw
    s = rows_flat.shape[0] // lev_tab.shape[0]
    mesh = plsc.VectorSubcoreMesh(core_axis_name="c", subcore_axis_name="s")

    @functools.partial(
        pl.kernel,
        mesh=mesh,
        out_type=[
            jax.ShapeDtypeStruct((b,), jnp.float32),
            jax.ShapeDtypeStruct((b,), jnp.float32),
            jax.ShapeDtypeStruct((s * b,), jnp.float32),
        ],
        scratch_types=[
            pltpu.VMEM((bpw,), jnp.int32),
            pltpu.VMEM((s, bpw), jnp.int32),
            pltpu.VMEM((bpw,), jnp.float32),
            pltpu.VMEM((bpw,), jnp.float32),
            pltpu.VMEM((s, bpw), jnp.float32),
            pltpu.SemaphoreType.DMA,
        ],
    )
    def gather_kernel(idx_hbm, lev_hbm, seas_hbm, flat_hbm,
                      lev_out, seas_out, rowst_out,
                      idx_v, idx2_v, lev_v, seas_v, rowst_v, sem):
        wid = lax.axis_index("s") * nc + lax.axis_index("c")
        base = wid * bpw
        pltpu.sync_copy(idx_hbm.at[pl.ds(base, bpw)], idx_v)
        for j in range(s):
            for ch in range(bpw // 16):
                idx2_v[j, pl.ds(ch * 16, 16)] = idx_v[pl.ds(ch * 16, 16)] * s + j
        copies = [
            pltpu.async_copy(lev_hbm.at[idx_v], lev_v, sem),
            pltpu.async_copy(seas_hbm.at[idx_v], seas_v, sem),
        ]
        for j in range(s):
            copies.append(
                pltpu.async_copy(flat_hbm.at[idx2_v.at[j]], rowst_v.at[j], sem))
        for c in copies:
            c.wait()
        pltpu.sync_copy(lev_v, lev_out.at[pl.ds(base, bpw)])
        pltpu.sync_copy(seas_v, seas_out.at[pl.ds(base, bpw)])
        stores = [
            pltpu.async_copy(rowst_v.at[j], rowst_out.at[pl.ds(j * b + base, bpw)], sem)
            for j in range(s)
        ]
        for c in stores:
            c.wait()

    return gather_kernel(idxs, lev_tab, seas_tab, rows_flat)


def _es_body(train_ref, lev_raw_ref, seas_raw_ref, seas0_ref,
             levs_ref, seas_ref, msq_ref, ll_ref):
    L = train_ref.shape[0]
    S = seas0_ref.shape[0]
    lev_sm = 1.0 / (1.0 + jnp.exp(-lev_raw_ref[...]))
    seas_sm = 1.0 / (1.0 + jnp.exp(-seas_raw_ref[...]))
    one_m_lev = 1.0 - lev_sm
    one_m_seas = 1.0 - seas_sm

    row0 = jnp.exp(seas0_ref[0])
    seas_ref[0] = row0
    for j in range(1, S):
        seas_ref[j] = jnp.exp(seas0_ref[j])
    seas_ref[S] = row0

    lev0 = train_ref[0] / row0
    levs_ref[0] = lev0

    def step(t, lev_prev):
        x = train_ref[t]
        s_t = seas_ref[t]
        new_lev = lev_sm * (x / s_t) + one_m_lev * lev_prev
        levs_ref[t] = new_lev
        seas_ref[t + S] = seas_sm * (x / new_lev) + one_m_seas * s_t
        return new_lev

    lax.fori_loop(1, L, step, lev0, unroll=8)

    ch = 32
    for c in range(0, L, ch):
        ll_ref[pl.ds(c, ch)] = jnp.log(levs_ref[pl.ds(c, ch)])

    dch = 30
    inv_b = 1.0 / (train_ref.shape[1] * train_ref.shape[2])
    for k in range(0, L - 2, dch):
        w = ll_ref[pl.ds(k, dch + 2)]
        d2 = w[2:] - 2.0 * w[1:-1] + w[:-2]
        msq_ref[0, pl.ds(k, dch)] = jnp.sum(d2 * d2, axis=(1, 2)) * inv_b


def _es_scan(train_t, lev_raw, seas_raw, seas0):
    L = train_t.shape[0]
    S = seas0.shape[0]
    return pl.pallas_call(
        _es_body,
        out_shape=[
            jax.ShapeDtypeStruct((L, 8, 128), jnp.float32),
            jax.ShapeDtypeStruct((S + L, 8, 128), jnp.float32),
            jax.ShapeDtypeStruct((1, 512), jnp.float32),
        ],
        scratch_shapes=[pltpu.VMEM((L, 8, 128), jnp.float32)],
    )(train_t, lev_raw, seas_raw, seas0)


def kernel(train, val, test, info_cat, idxs, init_lev_sms, init_seas_sms, init_seasonalities):
    B, L = train.shape
    S = init_seasonalities.shape[1]
    lev_g, seas_g, rowst_g = _sc_gather(
        idxs.astype(jnp.int32), init_lev_sms, init_seas_sms,
        init_seasonalities.reshape(-1))
    train_t = train.T.reshape(L, B // 128, 128)
    levs3, seas3, msq = _es_scan(
        train_t,
        lev_g.reshape(B // 128, 128),
        seas_g.reshape(B // 128, 128),
        rowst_g.reshape(S, B // 128, 128),
    )
    return levs3.reshape(L, B), seas3.reshape(S + L, B), msq[0, :L - 2]


# speedup vs baseline: 36.1526x; 1.0043x over previous
import functools

import jax
import jax.numpy as jnp
from jax import lax
from jax.experimental import pallas as pl
from jax.experimental.pallas import tpu as pltpu
from jax.experimental.pallas import tpu_sc as plsc


def _sc_gather(idxs, lev_tab, seas_tab, rows_flat):
    info = plsc.get_sparse_core_info()
    nc = info.num_cores
    nw = nc * info.num_subcores
    b = idxs.shape[0]
    bpw = b // nw
    s = rows_flat.shape[0] // lev_tab.shape[0]
    mesh = plsc.VectorSubcoreMesh(core_axis_name="c", subcore_axis_name="s")

    @functools.partial(
        pl.kernel,
        mesh=mesh,
        out_type=[
            jax.ShapeDtypeStruct((b,), jnp.float32),
            jax.ShapeDtypeStruct((b,), jnp.float32),
            jax.ShapeDtypeStruct((s * b,), jnp.float32),
        ],
        scratch_types=[
            pltpu.VMEM((bpw,), jnp.int32),
            pltpu.VMEM((s, bpw), jnp.int32),
            pltpu.VMEM((bpw,), jnp.float32),
            pltpu.VMEM((bpw,), jnp.float32),
            pltpu.VMEM((s, bpw), jnp.float32),
            pltpu.SemaphoreType.DMA,
        ],
    )
    def gather_kernel(idx_hbm, lev_hbm, seas_hbm, flat_hbm,
                      lev_out, seas_out, rowst_out,
                      idx_v, idx2_v, lev_v, seas_v, rowst_v, sem):
        wid = lax.axis_index("s") * nc + lax.axis_index("c")
        base = wid * bpw
        pltpu.sync_copy(idx_hbm.at[pl.ds(base, bpw)], idx_v)
        for j in range(s):
            for ch in range(bpw // 16):
                idx2_v[j, pl.ds(ch * 16, 16)] = idx_v[pl.ds(ch * 16, 16)] * s + j
        copies = [
            pltpu.async_copy(lev_hbm.at[idx_v], lev_v, sem),
            pltpu.async_copy(seas_hbm.at[idx_v], seas_v, sem),
        ]
        for j in range(s):
            copies.append(
                pltpu.async_copy(flat_hbm.at[idx2_v.at[j]], rowst_v.at[j], sem))
        for c in copies:
            c.wait()
        pltpu.sync_copy(lev_v, lev_out.at[pl.ds(base, bpw)])
        pltpu.sync_copy(seas_v, seas_out.at[pl.ds(base, bpw)])
        stores = [
            pltpu.async_copy(rowst_v.at[j], rowst_out.at[pl.ds(j * b + base, bpw)], sem)
            for j in range(s)
        ]
        for c in stores:
            c.wait()

    return gather_kernel(idxs, lev_tab, seas_tab, rows_flat)


def _es_body(train_ref, lev_raw_ref, seas_raw_ref, seas0_ref,
             levs_ref, seas_ref, msq_ref, tt_ref, ll_ref):
    L = tt_ref.shape[0]
    S = seas0_ref.shape[0]

    for j in range(L // 128):
        for i in range(train_ref.shape[0]):
            blk = train_ref[i, :, pl.ds(j * 128, 128)]
            tt_ref[pl.ds(j * 128, 128), i, :] = blk.T

    lev_sm = 1.0 / (1.0 + jnp.exp(-lev_raw_ref[...]))
    seas_sm = 1.0 / (1.0 + jnp.exp(-seas_raw_ref[...]))
    one_m_lev = 1.0 - lev_sm
    one_m_seas = 1.0 - seas_sm

    row0 = jnp.exp(seas0_ref[0])
    seas_ref[0] = row0
    for j in range(1, S):
        seas_ref[j] = jnp.exp(seas0_ref[j])
    seas_ref[S] = row0

    lev0 = tt_ref[0] / row0
    levs_ref[0] = lev0

    def block(t0, lev_prev, nb):
        x_blk = tt_ref[pl.ds(t0, nb)]
        s_blk = seas_ref[pl.ds(t0, nb)]
        aq_blk = lev_sm[None] * (x_blk / s_blk)
        levs = []
        for j in range(nb):
            lev_prev = aq_blk[j] + one_m_lev * lev_prev
            levs.append(lev_prev)
        lev_blk = jnp.stack(levs, axis=0)
        levs_ref[pl.ds(t0, nb)] = lev_blk
        seas_ref[pl.ds(t0 + S, nb)] = (
            seas_sm[None] * (x_blk / lev_blk) + one_m_seas[None] * s_blk)
        return lev_prev

    nfull = (L - 1) // S
    lev_prev = lax.fori_loop(
        0, nfull, lambda k, c: block(1 + k * S, c, S), lev0)
    tail = (L - 1) - nfull * S
    if tail:
        block(1 + nfull * S, lev_prev, tail)

    ch = 32
    for c in range(0, L, ch):
        ll_ref[pl.ds(c, ch)] = jnp.log(levs_ref[pl.ds(c, ch)])

    dch = 30
    inv_b = 1.0 / (tt_ref.shape[1] * tt_ref.shape[2])
    for k in range(0, L - 2, dch):
        w = ll_ref[pl.ds(k, dch + 2)]
        d2 = w[2:] - 2.0 * w[1:-1] + w[:-2]
        msq_ref[0, pl.ds(k, dch)] = jnp.sum(d2 * d2, axis=(1, 2)) * inv_b


def _es_scan(train_r, lev_raw, seas_raw, seas0):
    L = train_r.shape[2]
    S = seas0.shape[0]
    return pl.pallas_call(
        _es_body,
        out_shape=[
            jax.ShapeDtypeStruct((L, 8, 128), jnp.float32),
            jax.ShapeDtypeStruct((S + L, 8, 128), jnp.float32),
            jax.ShapeDtypeStruct((1, 512), jnp.float32),
        ],
        scratch_shapes=[pltpu.VMEM((L, 8, 128), jnp.float32),
                        pltpu.VMEM((L, 8, 128), jnp.float32)],
    )(train_r, lev_raw, seas_raw, seas0)


def kernel(train, val, test, info_cat, idxs, init_lev_sms, init_seas_sms, init_seasonalities):
    B, L = train.shape
    S = init_seasonalities.shape[1]
    lev_g, seas_g, rowst_g = _sc_gather(
        idxs.astype(jnp.int32), init_lev_sms, init_seas_sms,
        init_seasonalities.reshape(-1))
    levs3, seas3, msq = _es_scan(
        train.reshape(B // 128, 128, L),
        lev_g.reshape(B // 128, 128),
        seas_g.reshape(B // 128, 128),
        rowst_g.reshape(S, B // 128, 128),
    )
    return levs3.reshape(L, B), seas3.reshape(S + L, B), msq[0, :L - 2]


# speedup vs baseline: 56.9578x; 1.5755x over previous
import functools

import jax
import jax.numpy as jnp
from jax import lax
from jax.experimental import pallas as pl
from jax.experimental.pallas import tpu as pltpu
from jax.experimental.pallas import tpu_sc as plsc


def _sc_gather(idxs, lev_tab, seas_tab, seas2d):
    info = plsc.get_sparse_core_info()
    nc = info.num_cores
    nw = nc * info.num_subcores
    b = idxs.shape[0]
    bpw = b // nw
    s = seas2d.shape[1]
    mesh = plsc.VectorSubcoreMesh(core_axis_name="c", subcore_axis_name="s")

    @functools.partial(
        pl.kernel,
        mesh=mesh,
        out_type=[
            jax.ShapeDtypeStruct((b,), jnp.float32),
            jax.ShapeDtypeStruct((b,), jnp.float32),
            jax.ShapeDtypeStruct((b, s), jnp.float32),
        ],
        scratch_types=[
            pltpu.VMEM((bpw,), jnp.int32),
            pltpu.VMEM((bpw,), jnp.float32),
            pltpu.VMEM((bpw,), jnp.float32),
            pltpu.VMEM((bpw, s), jnp.float32),
            pltpu.SemaphoreType.DMA,
        ],
    )
    def gather_kernel(idx_hbm, lev_hbm, seas_hbm, tab_hbm,
                      lev_out, seas_out, rows_out,
                      idx_v, lev_v, seas_v, rows_v, sem):
        wid = lax.axis_index("s") * nc + lax.axis_index("c")
        base = wid * bpw
        pltpu.sync_copy(idx_hbm.at[pl.ds(base, bpw)], idx_v)
        copies = [
            pltpu.async_copy(lev_hbm.at[idx_v], lev_v, sem),
            pltpu.async_copy(seas_hbm.at[idx_v], seas_v, sem),
        ]
        for ch in range(bpw // 16):
            vec = idx_v[pl.ds(ch * 16, 16)]
            for t in range(16):
                copies.append(
                    pltpu.async_copy(tab_hbm.at[vec[t]],
                                     rows_v.at[ch * 16 + t], sem))
        for c in copies:
            c.wait()
        pltpu.sync_copy(lev_v, lev_out.at[pl.ds(base, bpw)])
        pltpu.sync_copy(seas_v, seas_out.at[pl.ds(base, bpw)])
        pltpu.sync_copy(rows_v, rows_out.at[pl.ds(base, bpw)])

    return gather_kernel(idxs, lev_tab, seas_tab, seas2d)


def _es_body(train_ref, lev_raw_ref, seas_raw_ref, rows_ref,
             levs_ref, seas_ref, msq_ref, tt_ref, ll_ref, s0t_ref):
    L = tt_ref.shape[0]
    S = s0t_ref.shape[0]

    for j in range(L // 128):
        for i in range(train_ref.shape[0]):
            blk = train_ref[i, :, pl.ds(j * 128, 128)]
            tt_ref[pl.ds(j * 128, 128), i, :] = blk.T

    for i in range(rows_ref.shape[0] // 128):
        s0t_ref[:, i, :] = rows_ref[pl.ds(i * 128, 128), :].T

    lev_sm = 1.0 / (1.0 + jnp.exp(-lev_raw_ref[...]))
    seas_sm = 1.0 / (1.0 + jnp.exp(-seas_raw_ref[...]))
    one_m_lev = 1.0 - lev_sm
    one_m_seas = 1.0 - seas_sm

    row0 = jnp.exp(s0t_ref[0])
    seas_ref[0] = row0
    for j in range(1, S):
        seas_ref[j] = jnp.exp(s0t_ref[j])
    seas_ref[S] = row0

    lev0 = tt_ref[0] / row0
    levs_ref[0] = lev0

    def block(t0, lev_prev, nb):
        x_blk = tt_ref[pl.ds(t0, nb)]
        s_blk = seas_ref[pl.ds(t0, nb)]
        aq_blk = lev_sm[None] * (x_blk / s_blk)
        levs = []
        for j in range(nb):
            lev_prev = aq_blk[j] + one_m_lev * lev_prev
            levs.append(lev_prev)
        lev_blk = jnp.stack(levs, axis=0)
        levs_ref[pl.ds(t0, nb)] = lev_blk
        seas_ref[pl.ds(t0 + S, nb)] = (
            seas_sm[None] * (x_blk / lev_blk) + one_m_seas[None] * s_blk)
        return lev_prev

    nfull = (L - 1) // S
    lev_prev = lax.fori_loop(
        0, nfull, lambda k, c: block(1 + k * S, c, S), lev0)
    tail = (L - 1) - nfull * S
    if tail:
        block(1 + nfull * S, lev_prev, tail)

    ch = 32
    for c in range(0, L, ch):
        ll_ref[pl.ds(c, ch)] = jnp.log(levs_ref[pl.ds(c, ch)])

    dch = 30
    inv_b = 1.0 / (tt_ref.shape[1] * tt_ref.shape[2])
    for k in range(0, L - 2, dch):
        w = ll_ref[pl.ds(k, dch + 2)]
        d2 = w[2:] - 2.0 * w[1:-1] + w[:-2]
        msq_ref[0, pl.ds(k, dch)] = jnp.sum(d2 * d2, axis=(1, 2)) * inv_b


def _es_scan(train_r, lev_raw, seas_raw, rows):
    L = train_r.shape[2]
    S = rows.shape[1]
    return pl.pallas_call(
        _es_body,
        out_shape=[
            jax.ShapeDtypeStruct((L, 8, 128), jnp.float32),
            jax.ShapeDtypeStruct((S + L, 8, 128), jnp.float32),
            jax.ShapeDtypeStruct((1, 512), jnp.float32),
        ],
        scratch_shapes=[pltpu.VMEM((L, 8, 128), jnp.float32),
                        pltpu.VMEM((L, 8, 128), jnp.float32),
                        pltpu.VMEM((S, 8, 128), jnp.float32)],
    )(train_r, lev_raw, seas_raw, rows)


def kernel(train, val, test, info_cat, idxs, init_lev_sms, init_seas_sms, init_seasonalities):
    B, L = train.shape
    S = init_seasonalities.shape[1]
    lev_g, seas_g, rows_g = _sc_gather(
        idxs.astype(jnp.int32), init_lev_sms, init_seas_sms,
        init_seasonalities)
    levs3, seas3, msq = _es_scan(
        train.reshape(B // 128, 128, L),
        lev_g.reshape(B // 128, 128),
        seas_g.reshape(B // 128, 128),
        rows_g,
    )
    return levs3.reshape(L, B), seas3.reshape(S + L, B), msq[0, :L - 2]
